# Initial kernel scaffold; baseline (speedup 1.0000x reference)
import jax
import jax.numpy as jnp
from jax.experimental import pallas as pl


def kernel(x, edge_index, W1, b1, Wr1, gamma1, beta1, W2, b2, Wr2):
    raise NotImplementedError("write your pallas kernel here")


# speedup vs baseline: 4.7687x; 4.7687x over previous
import functools

import jax
import jax.numpy as jnp
from jax import lax
from jax.experimental import pallas as pl
from jax.experimental.pallas import tpu as pltpu
from jax.experimental.pallas import tpu_sc as plsc

N_NODES = 10000
N_EDGES = 320000
D_IN = 128
D_HALF = 64
D_HID = 256
EPS = 1e-5

NC = 2
NS = 16
CH = 80
N_PAD = 10240
RPT = N_PAD // NS
CW = 16


def _init_spmem(zeros_hbm, buf, acc, r0):
    def body(k, carry):
        rr = r0 + k * CH
        pltpu.sync_copy(zeros_hbm.at[pl.ds(rr, CH)], buf)
        pltpu.sync_copy(buf, acc.at[pl.ds(rr, CH)])
        return carry
    lax.fori_loop(0, RPT // CH, body, 0)


def _writeout_spmem(acc, buf, out_hbm, cid, r0):
    def body(k, carry):
        rr = r0 + k * CH
        pltpu.sync_copy(acc.at[pl.ds(rr, CH)], buf)
        pltpu.sync_copy(buf, out_hbm.at[cid, pl.ds(rr, CH)])
        return carry
    lax.fori_loop(0, RPT // CH, body, 0)


@functools.cache
def _sc_kernels():
    mesh = plsc.VectorSubcoreMesh(core_axis_name="c", subcore_axis_name="s",
                                  num_cores=NC, num_subcores=NS)
    params = pltpu.CompilerParams(use_tc_tiling_on_sc=False)

    epw = N_EDGES // (NC * NS)

    @functools.partial(
        pl.kernel,
        out_type=jax.ShapeDtypeStruct((NC, N_PAD, CW), jnp.float32),
        mesh=mesh,
        compiler_params=params,
        scratch_types=[
            pltpu.VMEM((CH,), jnp.int32),
            pltpu.VMEM((CH, CW), jnp.float32),
            pltpu.VMEM_SHARED((N_PAD, CW), jnp.float32),
        ],
    )
    def sc_count(dst, zcnt, ones_h, cnt_out, didx, ones_v, cacc):
        cid = lax.axis_index("c")
        sid = lax.axis_index("s")
        r0 = sid * RPT
        _init_spmem(zcnt, ones_v, cacc, r0)
        pltpu.sync_copy(ones_h, ones_v)
        plsc.subcore_barrier()
        e0 = (cid * NS + sid) * epw

        def body(j, carry):
            pltpu.sync_copy(dst.at[pl.ds(e0 + j * CH, CH)], didx)
            pltpu.sync_copy(ones_v, cacc.at[didx], add=True)
            return carry
        lax.fori_loop(0, epw // CH, body, 0)
        plsc.subcore_barrier()
        _writeout_spmem(cacc, ones_v, cnt_out, cid, r0)

    @functools.partial(
        pl.kernel,
        out_type=jax.ShapeDtypeStruct((NC, N_PAD, D_HALF), jnp.float32),
        mesh=mesh,
        compiler_params=params,
        scratch_types=[
            pltpu.VMEM((CH,), jnp.int32),
            pltpu.VMEM((CH,), jnp.int32),
            pltpu.VMEM((CH, D_HALF), jnp.float32),
            pltpu.VMEM_SHARED((N_PAD, D_HALF), jnp.float32),
            pltpu.SemaphoreType.DMA,
        ],
    )
    def sc_segsum_x(xs, src, dst, zrow, sum_out, sidx, didx, rows, acc, sem):
        cid = lax.axis_index("c")
        sid = lax.axis_index("s")
        r0 = sid * RPT
        _init_spmem(zrow, rows, acc, r0)
        plsc.subcore_barrier()
        e0 = sid * (N_EDGES // NS)
        off = cid * N_NODES

        def body(j, carry):
            base = e0 + j * CH
            pltpu.sync_copy(src.at[pl.ds(base, CH)], sidx)
            pltpu.sync_copy(dst.at[pl.ds(base, CH)], didx)
            for k in range(CH // 16):
                sl = pl.ds(k * 16, 16)
                sidx[sl] = sidx[sl] + off
            pltpu.async_copy(xs.at[sidx], rows, sem).wait()
            pltpu.sync_copy(rows, acc.at[didx], add=True)
            return carry
        lax.fori_loop(0, (N_EDGES // NS) // CH, body, 0)
        plsc.subcore_barrier()
        _writeout_spmem(acc, rows, sum_out, cid, r0)

    @functools.partial(
        pl.kernel,
        out_type=jax.ShapeDtypeStruct((NC, N_PAD, CW), jnp.float32),
        mesh=mesh,
        compiler_params=params,
        scratch_types=[
            pltpu.VMEM((CH,), jnp.int32),
            pltpu.VMEM((CH,), jnp.int32),
            pltpu.VMEM((CH, CW), jnp.float32),
            pltpu.VMEM_SHARED((N_PAD, CW), jnp.float32),
            pltpu.SemaphoreType.DMA,
        ],
    )
    def sc_segsum_p(p, src, dst, zcnt, sum_out, sidx, didx, rows, acc, sem):
        cid = lax.axis_index("c")
        sid = lax.axis_index("s")
        r0 = sid * RPT
        _init_spmem(zcnt, rows, acc, r0)
        plsc.subcore_barrier()
        e0 = (cid * NS + sid) * epw

        def body(j, carry):
            base = e0 + j * CH
            pltpu.sync_copy(src.at[pl.ds(base, CH)], sidx)
            pltpu.sync_copy(dst.at[pl.ds(base, CH)], didx)
            pltpu.async_copy(p.at[sidx], rows, sem).wait()
            pltpu.sync_copy(rows, acc.at[didx], add=True)
            return carry
        lax.fori_loop(0, epw // CH, body, 0)
        plsc.subcore_barrier()
        _writeout_spmem(acc, rows, sum_out, cid, r0)

    return sc_count, sc_segsum_x, sc_segsum_p


BR = 1000
NB = N_NODES // BR


def _b1_body(sl, sr, c0, c1, x, w1tl, w1tr, wr1t, b1, t_out, stats_out,
             stat_acc):
    i = pl.program_id(0)
    cnt = jnp.maximum(jnp.max(c0[...] + c1[...], axis=1, keepdims=True), 1.0)
    inv = 1.0 / cnt
    t = (jnp.dot(sl[...] * inv, w1tl[...], preferred_element_type=jnp.float32)
         + jnp.dot(sr[...] * inv, w1tr[...], preferred_element_type=jnp.float32)
         + jnp.dot(x[...], wr1t[...], preferred_element_type=jnp.float32)
         + b1[...])
    t_out[...] = t

    @pl.when(i == 0)
    def _():
        stat_acc[...] = jnp.zeros_like(stat_acc)

    stat_acc[0:1, :] += jnp.sum(t, axis=0, keepdims=True)
    stat_acc[1:2, :] += jnp.sum(t * t, axis=0, keepdims=True)

    @pl.when(i == NB - 1)
    def _():
        stats_out[...] = stat_acc[...]


def _tc_b1(sl, sr, c0, c1, x, w1tl, w1tr, wr1t, b1):
    return pl.pallas_call(
        _b1_body,
        grid=(NB,),
        in_specs=[
            pl.BlockSpec((BR, D_HALF), lambda i: (i, 0)),
            pl.BlockSpec((BR, D_HALF), lambda i: (i, 0)),
            pl.BlockSpec((BR, CW), lambda i: (i, 0)),
            pl.BlockSpec((BR, CW), lambda i: (i, 0)),
            pl.BlockSpec((BR, D_IN), lambda i: (i, 0)),
            pl.BlockSpec((D_HALF, D_HID), lambda i: (0, 0)),
            pl.BlockSpec((D_HALF, D_HID), lambda i: (0, 0)),
            pl.BlockSpec((D_IN, D_HID), lambda i: (0, 0)),
            pl.BlockSpec((1, D_HID), lambda i: (0, 0)),
        ],
        out_specs=[
            pl.BlockSpec((BR, D_HID), lambda i: (i, 0)),
            pl.BlockSpec((8, D_HID), lambda i: (0, 0)),
        ],
        out_shape=[
            jax.ShapeDtypeStruct((N_NODES, D_HID), jnp.float32),
            jax.ShapeDtypeStruct((8, D_HID), jnp.float32),
        ],
        scratch_shapes=[pltpu.VMEM((8, D_HID), jnp.float32)],
    )(sl, sr, c0, c1, x, w1tl, w1tr, wr1t, b1)


def _b2_body(t, stats, gamma, beta, w2t, wr2t, p_out, q_out):
    inv_n = 1.0 / N_NODES
    mu = stats[0:1, :] * inv_n
    var = stats[1:2, :] * inv_n - mu * mu
    tn = (t[...] - mu) * lax.rsqrt(var + EPS) * gamma[...] + beta[...]
    h = jnp.where(tn > 0, tn, jnp.exp(jnp.minimum(tn, 0.0)) - 1.0)
    p_out[...] = jnp.dot(h, w2t[...], preferred_element_type=jnp.float32)
    q_out[...] = jnp.dot(h, wr2t[...], preferred_element_type=jnp.float32)


def _tc_b2(t, stats, gamma, beta, w2t, wr2t):
    return pl.pallas_call(
        _b2_body,
        grid=(NB,),
        in_specs=[
            pl.BlockSpec((BR, D_HID), lambda i: (i, 0)),
            pl.BlockSpec((8, D_HID), lambda i: (0, 0)),
            pl.BlockSpec((1, D_HID), lambda i: (0, 0)),
            pl.BlockSpec((1, D_HID), lambda i: (0, 0)),
            pl.BlockSpec((D_HID, CW), lambda i: (0, 0)),
            pl.BlockSpec((D_HID, CW), lambda i: (0, 0)),
        ],
        out_specs=[
            pl.BlockSpec((BR, CW), lambda i: (i, 0)),
            pl.BlockSpec((BR, CW), lambda i: (i, 0)),
        ],
        out_shape=[
            jax.ShapeDtypeStruct((N_NODES, CW), jnp.float32),
            jax.ShapeDtypeStruct((N_NODES, CW), jnp.float32),
        ],
    )(t, stats, gamma, beta, w2t, wr2t)


def _d_body(p0, p1, c0, c1, q, b2p, out):
    cnt = jnp.maximum(jnp.max(c0[...] + c1[...], axis=1, keepdims=True), 1.0)
    z = (p0[...] + p1[...]) * (1.0 / cnt) + b2p[...] + q[...]
    colmask = lax.broadcasted_iota(jnp.int32, z.shape, 1) < 2
    zm = jnp.where(colmask, z, -1e30)
    m = jnp.max(zm, axis=1, keepdims=True)
    ez = jnp.where(colmask, jnp.exp(z - m), 0.0)
    lse = m + jnp.log(jnp.sum(ez, axis=1, keepdims=True))
    out[...] = z - lse


def _tc_d(p0, p1, c0, c1, q, b2p):
    return pl.pallas_call(
        _d_body,
        grid=(NB,),
        in_specs=[pl.BlockSpec((BR, CW), lambda i: (i, 0))] * 5
        + [pl.BlockSpec((1, CW), lambda i: (0, 0))],
        out_specs=pl.BlockSpec((BR, CW), lambda i: (i, 0)),
        out_shape=jax.ShapeDtypeStruct((N_NODES, CW), jnp.float32),
    )(p0, p1, c0, c1, q, b2p)


def kernel(x, edge_index, W1, b1, Wr1, gamma1, beta1, W2, b2, Wr2):
    src = edge_index[0]
    dst = edge_index[1]

    zrow = jnp.zeros((N_PAD, D_HALF), jnp.float32)
    zcnt = jnp.zeros((N_PAD, CW), jnp.float32)
    ones_h = jnp.ones((CH, CW), jnp.float32)
    xs = jnp.concatenate([x[:, :D_HALF], x[:, D_HALF:]], axis=0)

    sc_count, sc_segsum_x, sc_segsum_p = _sc_kernels()
    cnts = sc_count(dst, zcnt, ones_h)
    sums = sc_segsum_x(xs, src, dst, zrow)

    w1t = W1.T
    b1r = b1.reshape(1, D_HID)
    t, stats = _tc_b1(sums[0], sums[1], cnts[0], cnts[1], x,
                      w1t[:D_HALF], w1t[D_HALF:], Wr1.T, b1r)

    w2t = jnp.zeros((D_HID, CW), jnp.float32).at[:, 0:2].set(W2.T)
    wr2t = jnp.zeros((D_HID, CW), jnp.float32).at[:, 0:2].set(Wr2.T)
    p, q = _tc_b2(t, stats, gamma1.reshape(1, D_HID), beta1.reshape(1, D_HID),
                  w2t, wr2t)

    psums = sc_segsum_p(p, src, dst, zcnt)

    b2p = jnp.zeros((1, CW), jnp.float32).at[0, 0:2].set(b2)
    out = _tc_d(psums[0], psums[1], cnts[0], cnts[1], q, b2p)
    return out[:, 0:2]


# speedup vs baseline: 6.8116x; 1.4284x over previous
import functools

import jax
import jax.numpy as jnp
from jax import lax
from jax.experimental import pallas as pl
from jax.experimental.pallas import tpu as pltpu
from jax.experimental.pallas import tpu_sc as plsc

N_NODES = 10000
N_EDGES = 320000
D_IN = 128
D_HALF = 64
D_HID = 256
EPS = 1e-5

NC = 2
NS = 16
CH = 80
N_PAD = 10240
RPT = N_PAD // NS
CW = 16


def _init_spmem(zeros_hbm, buf, acc, r0):
    def body(k, carry):
        rr = r0 + k * CH
        pltpu.sync_copy(zeros_hbm.at[pl.ds(rr, CH)], buf)
        pltpu.sync_copy(buf, acc.at[pl.ds(rr, CH)])
        return carry
    lax.fori_loop(0, RPT // CH, body, 0)


def _writeout_spmem(acc, buf, out_hbm, cid, r0):
    def body(k, carry):
        rr = r0 + k * CH
        pltpu.sync_copy(acc.at[pl.ds(rr, CH)], buf)
        pltpu.sync_copy(buf, out_hbm.at[cid, pl.ds(rr, CH)])
        return carry
    lax.fori_loop(0, RPT // CH, body, 0)


@functools.cache
def _sc_kernels():
    mesh = plsc.VectorSubcoreMesh(core_axis_name="c", subcore_axis_name="s",
                                  num_cores=NC, num_subcores=NS)
    params = pltpu.CompilerParams(use_tc_tiling_on_sc=False)

    epw = N_EDGES // (NC * NS)

    @functools.partial(
        pl.kernel,
        out_type=jax.ShapeDtypeStruct((NC, N_PAD, CW), jnp.float32),
        mesh=mesh,
        compiler_params=params,
        scratch_types=[
            pltpu.VMEM((CH,), jnp.int32),
            pltpu.VMEM((CH, CW), jnp.float32),
            pltpu.VMEM_SHARED((N_PAD, CW), jnp.float32),
        ],
    )
    def sc_count(dst, zcnt, ones_h, cnt_out, didx, ones_v, cacc):
        cid = lax.axis_index("c")
        sid = lax.axis_index("s")
        r0 = sid * RPT
        _init_spmem(zcnt, ones_v, cacc, r0)
        pltpu.sync_copy(ones_h, ones_v)
        plsc.subcore_barrier()
        e0 = (cid * NS + sid) * epw

        def body(j, carry):
            pltpu.sync_copy(dst.at[pl.ds(e0 + j * CH, CH)], didx)
            pltpu.sync_copy(ones_v, cacc.at[didx], add=True)
            return carry
        lax.fori_loop(0, epw // CH, body, 0)
        plsc.subcore_barrier()
        _writeout_spmem(cacc, ones_v, cnt_out, cid, r0)

    @functools.partial(
        pl.kernel,
        out_type=jax.ShapeDtypeStruct((NC, N_PAD, D_HALF), jnp.float32),
        mesh=mesh,
        compiler_params=params,
        scratch_types=[
            pltpu.VMEM((CH,), jnp.int32),
            pltpu.VMEM((CH,), jnp.int32),
            pltpu.VMEM((CH,), jnp.int32),
            pltpu.VMEM((CH,), jnp.int32),
            pltpu.VMEM((CH, D_HALF), jnp.float32),
            pltpu.VMEM((CH, D_HALF), jnp.float32),
            pltpu.VMEM_SHARED((N_PAD, D_HALF), jnp.float32),
            pltpu.SemaphoreType.DMA,
            pltpu.SemaphoreType.DMA,
        ],
    )
    def sc_segsum_x(xs, src, dst, zrow, sum_out,
                    sidx0, sidx1, didx0, didx1, rows0, rows1, acc,
                    sem0, sem1):
        cid = lax.axis_index("c")
        sid = lax.axis_index("s")
        r0 = sid * RPT
        _init_spmem(zrow, rows0, acc, r0)
        plsc.subcore_barrier()
        niter = (N_EDGES // NS) // CH
        e0 = sid * (N_EDGES // NS)
        off = cid * N_NODES
        bufs = ((sidx0, didx0, rows0, sem0), (sidx1, didx1, rows1, sem1))

        def start(j, b):
            sidx, didx, rows, sem = bufs[b]
            base = e0 + j * CH
            pltpu.sync_copy(src.at[pl.ds(base, CH)], sidx)
            pltpu.sync_copy(dst.at[pl.ds(base, CH)], didx)
            for k in range(CH // 16):
                sl = pl.ds(k * 16, 16)
                sidx[sl] = sidx[sl] + off
            pltpu.async_copy(xs.at[sidx], rows, sem)

        def finish(b):
            sidx, didx, rows, sem = bufs[b]
            pltpu.make_async_copy(xs.at[sidx], rows, sem).wait()
            pltpu.sync_copy(rows, acc.at[didx], add=True)

        start(0, 0)

        def body(jj, carry):
            j = 2 * jj
            start(j + 1, 1)
            finish(0)

            @pl.when(j + 2 < niter)
            def _():
                start(j + 2, 0)
            finish(1)
            return carry
        lax.fori_loop(0, niter // 2, body, 0)
        if niter % 2 == 1:
            finish(0)
        plsc.subcore_barrier()
        _writeout_spmem(acc, rows0, sum_out, cid, r0)

    @functools.partial(
        pl.kernel,
        out_type=jax.ShapeDtypeStruct((NC, N_PAD, CW), jnp.float32),
        mesh=mesh,
        compiler_params=params,
        scratch_types=[
            pltpu.VMEM((CH,), jnp.int32),
            pltpu.VMEM((CH,), jnp.int32),
            pltpu.VMEM((CH,), jnp.int32),
            pltpu.VMEM((CH,), jnp.int32),
            pltpu.VMEM((CH, CW), jnp.float32),
            pltpu.VMEM((CH, CW), jnp.float32),
            pltpu.VMEM_SHARED((N_PAD, CW), jnp.float32),
            pltpu.SemaphoreType.DMA,
            pltpu.SemaphoreType.DMA,
        ],
    )
    def sc_segsum_p(p, src, dst, zcnt, sum_out,
                    sidx0, sidx1, didx0, didx1, rows0, rows1, acc,
                    sem0, sem1):
        cid = lax.axis_index("c")
        sid = lax.axis_index("s")
        r0 = sid * RPT
        _init_spmem(zcnt, rows0, acc, r0)
        plsc.subcore_barrier()
        niter = epw // CH
        e0 = (cid * NS + sid) * epw
        bufs = ((sidx0, didx0, rows0, sem0), (sidx1, didx1, rows1, sem1))

        def start(j, b):
            sidx, didx, rows, sem = bufs[b]
            base = e0 + j * CH
            pltpu.sync_copy(src.at[pl.ds(base, CH)], sidx)
            pltpu.sync_copy(dst.at[pl.ds(base, CH)], didx)
            pltpu.async_copy(p.at[sidx], rows, sem)

        def finish(b):
            sidx, didx, rows, sem = bufs[b]
            pltpu.make_async_copy(p.at[sidx], rows, sem).wait()
            pltpu.sync_copy(rows, acc.at[didx], add=True)

        start(0, 0)

        def body(jj, carry):
            j = 2 * jj
            start(j + 1, 1)
            finish(0)

            @pl.when(j + 2 < niter)
            def _():
                start(j + 2, 0)
            finish(1)
            return carry
        lax.fori_loop(0, niter // 2, body, 0)
        if (epw // CH) % 2 == 1:
            finish(0)
        plsc.subcore_barrier()
        _writeout_spmem(acc, rows0, sum_out, cid, r0)

    return sc_count, sc_segsum_x, sc_segsum_p


BR = 1000
NB = N_NODES // BR


def _b1_body(sl, sr, c0, c1, x, w1tl, w1tr, wr1t, b1, t_out, stats_out,
             stat_acc):
    i = pl.program_id(0)
    cnt = jnp.maximum(jnp.max(c0[...] + c1[...], axis=1, keepdims=True), 1.0)
    inv = 1.0 / cnt
    t = (jnp.dot(sl[...] * inv, w1tl[...], preferred_element_type=jnp.float32)
         + jnp.dot(sr[...] * inv, w1tr[...], preferred_element_type=jnp.float32)
         + jnp.dot(x[...], wr1t[...], preferred_element_type=jnp.float32)
         + b1[...])
    t_out[...] = t

    @pl.when(i == 0)
    def _():
        stat_acc[...] = jnp.zeros_like(stat_acc)

    stat_acc[0:1, :] += jnp.sum(t, axis=0, keepdims=True)
    stat_acc[1:2, :] += jnp.sum(t * t, axis=0, keepdims=True)

    @pl.when(i == NB - 1)
    def _():
        stats_out[...] = stat_acc[...]


def _tc_b1(sl, sr, c0, c1, x, w1tl, w1tr, wr1t, b1):
    return pl.pallas_call(
        _b1_body,
        grid=(NB,),
        in_specs=[
            pl.BlockSpec((BR, D_HALF), lambda i: (i, 0)),
            pl.BlockSpec((BR, D_HALF), lambda i: (i, 0)),
            pl.BlockSpec((BR, CW), lambda i: (i, 0)),
            pl.BlockSpec((BR, CW), lambda i: (i, 0)),
            pl.BlockSpec((BR, D_IN), lambda i: (i, 0)),
            pl.BlockSpec((D_HALF, D_HID), lambda i: (0, 0)),
            pl.BlockSpec((D_HALF, D_HID), lambda i: (0, 0)),
            pl.BlockSpec((D_IN, D_HID), lambda i: (0, 0)),
            pl.BlockSpec((1, D_HID), lambda i: (0, 0)),
        ],
        out_specs=[
            pl.BlockSpec((BR, D_HID), lambda i: (i, 0)),
            pl.BlockSpec((8, D_HID), lambda i: (0, 0)),
        ],
        out_shape=[
            jax.ShapeDtypeStruct((N_NODES, D_HID), jnp.float32),
            jax.ShapeDtypeStruct((8, D_HID), jnp.float32),
        ],
        scratch_shapes=[pltpu.VMEM((8, D_HID), jnp.float32)],
    )(sl, sr, c0, c1, x, w1tl, w1tr, wr1t, b1)


def _b2_body(t, stats, gamma, beta, w2t, wr2t, p_out, q_out):
    inv_n = 1.0 / N_NODES
    mu = stats[0:1, :] * inv_n
    var = stats[1:2, :] * inv_n - mu * mu
    tn = (t[...] - mu) * lax.rsqrt(var + EPS) * gamma[...] + beta[...]
    h = jnp.where(tn > 0, tn, jnp.exp(jnp.minimum(tn, 0.0)) - 1.0)
    p_out[...] = jnp.dot(h, w2t[...], preferred_element_type=jnp.float32)
    q_out[...] = jnp.dot(h, wr2t[...], preferred_element_type=jnp.float32)


def _tc_b2(t, stats, gamma, beta, w2t, wr2t):
    return pl.pallas_call(
        _b2_body,
        grid=(NB,),
        in_specs=[
            pl.BlockSpec((BR, D_HID), lambda i: (i, 0)),
            pl.BlockSpec((8, D_HID), lambda i: (0, 0)),
            pl.BlockSpec((1, D_HID), lambda i: (0, 0)),
            pl.BlockSpec((1, D_HID), lambda i: (0, 0)),
            pl.BlockSpec((D_HID, CW), lambda i: (0, 0)),
            pl.BlockSpec((D_HID, CW), lambda i: (0, 0)),
        ],
        out_specs=[
            pl.BlockSpec((BR, CW), lambda i: (i, 0)),
            pl.BlockSpec((BR, CW), lambda i: (i, 0)),
        ],
        out_shape=[
            jax.ShapeDtypeStruct((N_NODES, CW), jnp.float32),
            jax.ShapeDtypeStruct((N_NODES, CW), jnp.float32),
        ],
    )(t, stats, gamma, beta, w2t, wr2t)


def _d_body(p0, p1, c0, c1, q, b2p, out):
    cnt = jnp.maximum(jnp.max(c0[...] + c1[...], axis=1, keepdims=True), 1.0)
    z = (p0[...] + p1[...]) * (1.0 / cnt) + b2p[...] + q[...]
    colmask = lax.broadcasted_iota(jnp.int32, z.shape, 1) < 2
    zm = jnp.where(colmask, z, -1e30)
    m = jnp.max(zm, axis=1, keepdims=True)
    ez = jnp.where(colmask, jnp.exp(z - m), 0.0)
    lse = m + jnp.log(jnp.sum(ez, axis=1, keepdims=True))
    out[...] = z - lse


def _tc_d(p0, p1, c0, c1, q, b2p):
    return pl.pallas_call(
        _d_body,
        grid=(NB,),
        in_specs=[pl.BlockSpec((BR, CW), lambda i: (i, 0))] * 5
        + [pl.BlockSpec((1, CW), lambda i: (0, 0))],
        out_specs=pl.BlockSpec((BR, CW), lambda i: (i, 0)),
        out_shape=jax.ShapeDtypeStruct((N_NODES, CW), jnp.float32),
    )(p0, p1, c0, c1, q, b2p)


def kernel(x, edge_index, W1, b1, Wr1, gamma1, beta1, W2, b2, Wr2):
    src = edge_index[0]
    dst = edge_index[1]

    zrow = jnp.zeros((N_PAD, D_HALF), jnp.float32)
    zcnt = jnp.zeros((N_PAD, CW), jnp.float32)
    ones_h = jnp.ones((CH, CW), jnp.float32)
    xs = jnp.concatenate([x[:, :D_HALF], x[:, D_HALF:]], axis=0)

    sc_count, sc_segsum_x, sc_segsum_p = _sc_kernels()
    cnts = sc_count(dst, zcnt, ones_h)
    sums = sc_segsum_x(xs, src, dst, zrow)

    w1t = W1.T
    b1r = b1.reshape(1, D_HID)
    t, stats = _tc_b1(sums[0], sums[1], cnts[0], cnts[1], x,
                      w1t[:D_HALF], w1t[D_HALF:], Wr1.T, b1r)

    w2t = jnp.zeros((D_HID, CW), jnp.float32).at[:, 0:2].set(W2.T)
    wr2t = jnp.zeros((D_HID, CW), jnp.float32).at[:, 0:2].set(Wr2.T)
    p, q = _tc_b2(t, stats, gamma1.reshape(1, D_HID), beta1.reshape(1, D_HID),
                  w2t, wr2t)

    psums = sc_segsum_p(p, src, dst, zcnt)

    b2p = jnp.zeros((1, CW), jnp.float32).at[0, 0:2].set(b2)
    out = _tc_d(psums[0], psums[1], cnts[0], cnts[1], q, b2p)
    return out[:, 0:2]


# speedup vs baseline: 9.9452x; 1.4600x over previous
import functools

import jax
import jax.numpy as jnp
from jax import lax
from jax.experimental import pallas as pl
from jax.experimental.pallas import tpu as pltpu
from jax.experimental.pallas import tpu_sc as plsc

N_NODES = 10000
N_EDGES = 320000
D_IN = 128
D_HALF = 64
D_HID = 256
EPS = 1e-5

NC = 2
NS = 16
CH = 80
N_PAD = 10240
RPT = N_PAD // NS
CW = 16


def _init_spmem(zeros_hbm, buf, acc, r0):
    def body(k, carry):
        rr = r0 + k * CH
        pltpu.sync_copy(zeros_hbm.at[pl.ds(rr, CH)], buf)
        pltpu.sync_copy(buf, acc.at[pl.ds(rr, CH)])
        return carry
    lax.fori_loop(0, RPT // CH, body, 0)


def _writeout_spmem(acc, buf, out_hbm, cid, r0):
    def body(k, carry):
        rr = r0 + k * CH
        pltpu.sync_copy(acc.at[pl.ds(rr, CH)], buf)
        pltpu.sync_copy(buf, out_hbm.at[cid, pl.ds(rr, CH)])
        return carry
    lax.fori_loop(0, RPT // CH, body, 0)


@functools.cache
def _sc_kernels():
    mesh = plsc.VectorSubcoreMesh(core_axis_name="c", subcore_axis_name="s",
                                  num_cores=NC, num_subcores=NS)
    params = pltpu.CompilerParams(use_tc_tiling_on_sc=False)

    epw = N_EDGES // (NC * NS)

    @functools.partial(
        pl.kernel,
        out_type=jax.ShapeDtypeStruct((NC, N_PAD, CW), jnp.float32),
        mesh=mesh,
        compiler_params=params,
        scratch_types=[
            pltpu.VMEM((CH,), jnp.int32),
            pltpu.VMEM((CH,), jnp.int32),
            pltpu.VMEM((CH, CW), jnp.float32),
            pltpu.VMEM_SHARED((N_PAD, CW), jnp.float32),
            pltpu.SemaphoreType.DMA,
            pltpu.SemaphoreType.DMA,
        ],
    )
    def sc_count(dst, zcnt, ones_h, cnt_out, didx0, didx1, ones_v, cacc,
                 dsem0, dsem1):
        cid = lax.axis_index("c")
        sid = lax.axis_index("s")
        r0 = sid * RPT
        _init_spmem(zcnt, ones_v, cacc, r0)
        pltpu.sync_copy(ones_h, ones_v)
        plsc.subcore_barrier()
        niter = epw // CH
        e0 = (cid * NS + sid) * epw
        bufs = ((didx0, dsem0), (didx1, dsem1))

        def load(j, b):
            didx, dsem = bufs[b]
            pltpu.async_copy(dst.at[pl.ds(e0 + j * CH, CH)], didx, dsem)

        def scat(j, b):
            didx, dsem = bufs[b]
            pltpu.make_async_copy(
                dst.at[pl.ds(e0 + j * CH, CH)], didx, dsem).wait()
            pltpu.sync_copy(ones_v, cacc.at[didx], add=True)

        load(0, 0)
        load(1, 1)

        def body(jj, carry):
            j = 2 * jj
            scat(j, 0)

            @pl.when(j + 2 < niter)
            def _():
                load(j + 2, 0)
            scat(j + 1, 1)

            @pl.when(j + 3 < niter)
            def _():
                load(j + 3, 1)
            return carry
        lax.fori_loop(0, niter // 2, body, 0)
        if niter % 2 == 1:
            scat(niter - 1, 0)
        plsc.subcore_barrier()
        _writeout_spmem(cacc, ones_v, cnt_out, cid, r0)

    @functools.partial(
        pl.kernel,
        out_type=jax.ShapeDtypeStruct((NC, N_PAD, D_HALF), jnp.float32),
        mesh=mesh,
        compiler_params=params,
        scratch_types=[
            pltpu.VMEM((2, CH), jnp.int32),
            pltpu.VMEM((2, CH), jnp.int32),
            pltpu.VMEM((CH,), jnp.int32),
            pltpu.VMEM((CH,), jnp.int32),
            pltpu.VMEM((CH, D_HALF), jnp.float32),
            pltpu.VMEM((CH, D_HALF), jnp.float32),
            pltpu.VMEM_SHARED((N_PAD, D_HALF), jnp.float32),
            pltpu.SemaphoreType.DMA,
            pltpu.SemaphoreType.DMA,
            pltpu.SemaphoreType.DMA,
            pltpu.SemaphoreType.DMA,
        ],
    )
    def sc_segsum_x(edges, xs, zrow, sum_out,
                    ebuf0, ebuf1, didx0, didx1, rows0, rows1, acc,
                    esem0, esem1, gsem0, gsem1):
        cid = lax.axis_index("c")
        sid = lax.axis_index("s")
        r0 = sid * RPT
        _init_spmem(zrow, rows0, acc, r0)
        plsc.subcore_barrier()
        niter = (N_EDGES // NS) // CH
        e0 = sid * (N_EDGES // NS)
        off = cid * N_NODES
        bufs = ((ebuf0, didx0, rows0, esem0, gsem0),
                (ebuf1, didx1, rows1, esem1, gsem1))

        def load(j, b):
            ebuf, _, _, esem, _ = bufs[b]
            pltpu.async_copy(edges.at[:, pl.ds(e0 + j * CH, CH)], ebuf, esem)

        def gather(j, b):
            ebuf, didx, rows, esem, gsem = bufs[b]
            pltpu.make_async_copy(
                edges.at[:, pl.ds(e0 + j * CH, CH)], ebuf, esem).wait()
            for k in range(CH // 16):
                sl = pl.ds(k * 16, 16)
                ebuf[0, sl] = ebuf[0, sl] + off
                didx[sl] = ebuf[1, sl]
            pltpu.async_copy(xs.at[ebuf.at[0]], rows, gsem)

        def wait_gather(b):
            ebuf, _, rows, _, gsem = bufs[b]
            pltpu.make_async_copy(xs.at[ebuf.at[0]], rows, gsem).wait()

        def scatter(b):
            _, didx, rows, _, _ = bufs[b]
            pltpu.sync_copy(rows, acc.at[didx], add=True)

        load(0, 0)
        load(1, 1)
        gather(0, 0)

        def body(jj, carry):
            j = 2 * jj
            gather(j + 1, 1)
            wait_gather(0)

            @pl.when(j + 2 < niter)
            def _():
                load(j + 2, 0)
            scatter(0)

            @pl.when(j + 2 < niter)
            def _():
                gather(j + 2, 0)
            wait_gather(1)

            @pl.when(j + 3 < niter)
            def _():
                load(j + 3, 1)
            scatter(1)
            return carry
        lax.fori_loop(0, niter // 2, body, 0)
        if niter % 2 == 1:
            wait_gather(0)
            scatter(0)
        plsc.subcore_barrier()
        _writeout_spmem(acc, rows0, sum_out, cid, r0)

    @functools.partial(
        pl.kernel,
        out_type=jax.ShapeDtypeStruct((NC, N_PAD, CW), jnp.float32),
        mesh=mesh,
        compiler_params=params,
        scratch_types=[
            pltpu.VMEM((2, CH), jnp.int32),
            pltpu.VMEM((2, CH), jnp.int32),
            pltpu.VMEM((CH,), jnp.int32),
            pltpu.VMEM((CH,), jnp.int32),
            pltpu.VMEM((CH, CW), jnp.float32),
            pltpu.VMEM((CH, CW), jnp.float32),
            pltpu.VMEM_SHARED((N_PAD, CW), jnp.float32),
            pltpu.SemaphoreType.DMA,
            pltpu.SemaphoreType.DMA,
            pltpu.SemaphoreType.DMA,
            pltpu.SemaphoreType.DMA,
        ],
    )
    def sc_segsum_p(edges, p, zcnt, sum_out,
                    ebuf0, ebuf1, didx0, didx1, rows0, rows1, acc,
                    esem0, esem1, gsem0, gsem1):
        cid = lax.axis_index("c")
        sid = lax.axis_index("s")
        r0 = sid * RPT
        _init_spmem(zcnt, rows0, acc, r0)
        plsc.subcore_barrier()
        niter = epw // CH
        e0 = (cid * NS + sid) * epw
        bufs = ((ebuf0, didx0, rows0, esem0, gsem0),
                (ebuf1, didx1, rows1, esem1, gsem1))

        def load(j, b):
            ebuf, _, _, esem, _ = bufs[b]
            pltpu.async_copy(edges.at[:, pl.ds(e0 + j * CH, CH)], ebuf, esem)

        def gather(j, b):
            ebuf, didx, rows, esem, gsem = bufs[b]
            pltpu.make_async_copy(
                edges.at[:, pl.ds(e0 + j * CH, CH)], ebuf, esem).wait()
            for k in range(CH // 16):
                sl = pl.ds(k * 16, 16)
                didx[sl] = ebuf[1, sl]
            pltpu.async_copy(p.at[ebuf.at[0]], rows, gsem)

        def wait_gather(b):
            ebuf, _, rows, _, gsem = bufs[b]
            pltpu.make_async_copy(p.at[ebuf.at[0]], rows, gsem).wait()

        def scatter(b):
            _, didx, rows, _, _ = bufs[b]
            pltpu.sync_copy(rows, acc.at[didx], add=True)

        load(0, 0)
        load(1, 1)
        gather(0, 0)

        def body(jj, carry):
            j = 2 * jj
            gather(j + 1, 1)
            wait_gather(0)

            @pl.when(j + 2 < niter)
            def _():
                load(j + 2, 0)
            scatter(0)

            @pl.when(j + 2 < niter)
            def _():
                gather(j + 2, 0)
            wait_gather(1)

            @pl.when(j + 3 < niter)
            def _():
                load(j + 3, 1)
            scatter(1)
            return carry
        lax.fori_loop(0, niter // 2, body, 0)
        if (epw // CH) % 2 == 1:
            wait_gather(0)
            scatter(0)
        plsc.subcore_barrier()
        _writeout_spmem(acc, rows0, sum_out, cid, r0)

    return sc_count, sc_segsum_x, sc_segsum_p


BR = 1000
NB = N_NODES // BR


def _b1_body(sl, sr, c0, c1, x, w1tl, w1tr, wr1t, b1, t_out, stats_out,
             stat_acc):
    i = pl.program_id(0)
    cnt = jnp.maximum(jnp.max(c0[...] + c1[...], axis=1, keepdims=True), 1.0)
    inv = 1.0 / cnt
    t = (jnp.dot(sl[...] * inv, w1tl[...], preferred_element_type=jnp.float32)
         + jnp.dot(sr[...] * inv, w1tr[...], preferred_element_type=jnp.float32)
         + jnp.dot(x[...], wr1t[...], preferred_element_type=jnp.float32)
         + b1[...])
    t_out[...] = t

    @pl.when(i == 0)
    def _():
        stat_acc[...] = jnp.zeros_like(stat_acc)

    stat_acc[0:1, :] += jnp.sum(t, axis=0, keepdims=True)
    stat_acc[1:2, :] += jnp.sum(t * t, axis=0, keepdims=True)

    @pl.when(i == NB - 1)
    def _():
        stats_out[...] = stat_acc[...]


def _tc_b1(sl, sr, c0, c1, x, w1tl, w1tr, wr1t, b1):
    return pl.pallas_call(
        _b1_body,
        grid=(NB,),
        in_specs=[
            pl.BlockSpec((BR, D_HALF), lambda i: (i, 0)),
            pl.BlockSpec((BR, D_HALF), lambda i: (i, 0)),
            pl.BlockSpec((BR, CW), lambda i: (i, 0)),
            pl.BlockSpec((BR, CW), lambda i: (i, 0)),
            pl.BlockSpec((BR, D_IN), lambda i: (i, 0)),
            pl.BlockSpec((D_HALF, D_HID), lambda i: (0, 0)),
            pl.BlockSpec((D_HALF, D_HID), lambda i: (0, 0)),
            pl.BlockSpec((D_IN, D_HID), lambda i: (0, 0)),
            pl.BlockSpec((1, D_HID), lambda i: (0, 0)),
        ],
        out_specs=[
            pl.BlockSpec((BR, D_HID), lambda i: (i, 0)),
            pl.BlockSpec((8, D_HID), lambda i: (0, 0)),
        ],
        out_shape=[
            jax.ShapeDtypeStruct((N_NODES, D_HID), jnp.float32),
            jax.ShapeDtypeStruct((8, D_HID), jnp.float32),
        ],
        scratch_shapes=[pltpu.VMEM((8, D_HID), jnp.float32)],
    )(sl, sr, c0, c1, x, w1tl, w1tr, wr1t, b1)


def _b2_body(t, stats, gamma, beta, w2t, wr2t, p_out, q_out):
    inv_n = 1.0 / N_NODES
    mu = stats[0:1, :] * inv_n
    var = stats[1:2, :] * inv_n - mu * mu
    tn = (t[...] - mu) * lax.rsqrt(var + EPS) * gamma[...] + beta[...]
    h = jnp.where(tn > 0, tn, jnp.exp(jnp.minimum(tn, 0.0)) - 1.0)
    p_out[...] = jnp.dot(h, w2t[...], preferred_element_type=jnp.float32)
    q_out[...] = jnp.dot(h, wr2t[...], preferred_element_type=jnp.float32)


def _tc_b2(t, stats, gamma, beta, w2t, wr2t):
    return pl.pallas_call(
        _b2_body,
        grid=(NB,),
        in_specs=[
            pl.BlockSpec((BR, D_HID), lambda i: (i, 0)),
            pl.BlockSpec((8, D_HID), lambda i: (0, 0)),
            pl.BlockSpec((1, D_HID), lambda i: (0, 0)),
            pl.BlockSpec((1, D_HID), lambda i: (0, 0)),
            pl.BlockSpec((D_HID, CW), lambda i: (0, 0)),
            pl.BlockSpec((D_HID, CW), lambda i: (0, 0)),
        ],
        out_specs=[
            pl.BlockSpec((BR, CW), lambda i: (i, 0)),
            pl.BlockSpec((BR, CW), lambda i: (i, 0)),
        ],
        out_shape=[
            jax.ShapeDtypeStruct((N_NODES, CW), jnp.float32),
            jax.ShapeDtypeStruct((N_NODES, CW), jnp.float32),
        ],
    )(t, stats, gamma, beta, w2t, wr2t)


def _d_body(p0, p1, c0, c1, q, b2p, out):
    cnt = jnp.maximum(jnp.max(c0[...] + c1[...], axis=1, keepdims=True), 1.0)
    z = (p0[...] + p1[...]) * (1.0 / cnt) + b2p[...] + q[...]
    colmask = lax.broadcasted_iota(jnp.int32, z.shape, 1) < 2
    zm = jnp.where(colmask, z, -1e30)
    m = jnp.max(zm, axis=1, keepdims=True)
    ez = jnp.where(colmask, jnp.exp(z - m), 0.0)
    lse = m + jnp.log(jnp.sum(ez, axis=1, keepdims=True))
    out[...] = z - lse


def _tc_d(p0, p1, c0, c1, q, b2p):
    return pl.pallas_call(
        _d_body,
        grid=(NB,),
        in_specs=[pl.BlockSpec((BR, CW), lambda i: (i, 0))] * 5
        + [pl.BlockSpec((1, CW), lambda i: (0, 0))],
        out_specs=pl.BlockSpec((BR, CW), lambda i: (i, 0)),
        out_shape=jax.ShapeDtypeStruct((N_NODES, CW), jnp.float32),
    )(p0, p1, c0, c1, q, b2p)


def kernel(x, edge_index, W1, b1, Wr1, gamma1, beta1, W2, b2, Wr2):
    dst = edge_index[1]

    zrow = jnp.zeros((N_PAD, D_HALF), jnp.float32)
    zcnt = jnp.zeros((N_PAD, CW), jnp.float32)
    ones_h = jnp.ones((CH, CW), jnp.float32)
    xs = jnp.concatenate([x[:, :D_HALF], x[:, D_HALF:]], axis=0)

    sc_count, sc_segsum_x, sc_segsum_p = _sc_kernels()
    cnts = sc_count(dst, zcnt, ones_h)
    sums = sc_segsum_x(edge_index, xs, zrow)

    w1t = W1.T
    b1r = b1.reshape(1, D_HID)
    t, stats = _tc_b1(sums[0], sums[1], cnts[0], cnts[1], x,
                      w1t[:D_HALF], w1t[D_HALF:], Wr1.T, b1r)

    w2t = jnp.zeros((D_HID, CW), jnp.float32).at[:, 0:2].set(W2.T)
    wr2t = jnp.zeros((D_HID, CW), jnp.float32).at[:, 0:2].set(Wr2.T)
    p, q = _tc_b2(t, stats, gamma1.reshape(1, D_HID), beta1.reshape(1, D_HID),
                  w2t, wr2t)

    psums = sc_segsum_p(edge_index, p, zcnt)

    b2p = jnp.zeros((1, CW), jnp.float32).at[0, 0:2].set(b2)
    out = _tc_d(psums[0], psums[1], cnts[0], cnts[1], q, b2p)
    return out[:, 0:2]
